# Initial kernel scaffold; baseline (speedup 1.0000x reference)
import jax
import jax.numpy as jnp
from jax.experimental import pallas as pl


def kernel(x, codebooks):
    raise NotImplementedError("write your pallas kernel here")


# speedup vs baseline: 1.5341x; 1.5341x over previous
import jax
import jax.numpy as jnp
from jax.experimental import pallas as pl
from jax.experimental.pallas import tpu as pltpu

_B, _D, _T = 16, 256, 2048
_N_Q, _BINS = 8, 1024


def _rvq_kernel(x_ref, cb_ref, q_ref, codes_ref, loss_ref):
    r = x_ref[0]
    quant = jnp.zeros_like(r)
    rn = jnp.sum(r * r, axis=0)
    tb = r.shape[1]
    losses = []
    for i in range(_N_Q):
        cb = cb_ref[i]
        cbn = jnp.sum(cb * cb, axis=1)
        s = jax.lax.dot_general(
            cb, r, (((1,), (0,)), ((), ())),
            preferred_element_type=jnp.float32)
        d = (rn[None, :] - 2.0 * s) + cbn[:, None]
        idx = jnp.argmin(d, axis=0)
        onehot = (jax.lax.broadcasted_iota(jnp.int32, (_BINS, tb), 0)
                  == idx[None, :]).astype(jnp.float32)
        q = jax.lax.dot_general(
            cb, onehot, (((0,), (0,)), ((), ())),
            preferred_element_type=jnp.float32,
            precision=jax.lax.Precision.HIGHEST)
        q_st = r + (q - r)
        losses.append(jnp.sum((q - r) ** 2))
        quant = quant + q_st
        r = r - q_st
        rn = jnp.sum(r * r, axis=0)
        codes_ref[0, i, :] = idx
    q_ref[0] = quant
    loss_ref[0, 0, :] = jnp.stack(losses)


def kernel(x, codebooks):
    grid = (_B,)
    q_bdt, codes_bqt, loss_bq = pl.pallas_call(
        _rvq_kernel,
        grid=grid,
        in_specs=[
            pl.BlockSpec((1, _D, _T), lambda b: (b, 0, 0)),
            pl.BlockSpec((_N_Q, _BINS, _D), lambda b: (0, 0, 0)),
        ],
        out_specs=[
            pl.BlockSpec((1, _D, _T), lambda b: (b, 0, 0)),
            pl.BlockSpec((1, _N_Q, _T), lambda b: (b, 0, 0)),
            pl.BlockSpec((1, 1, _N_Q), lambda b: (b, 0, 0)),
        ],
        out_shape=[
            jax.ShapeDtypeStruct((_B, _D, _T), jnp.float32),
            jax.ShapeDtypeStruct((_B, _N_Q, _T), jnp.int32),
            jax.ShapeDtypeStruct((_B, 1, _N_Q), jnp.float32),
        ],
        compiler_params=pltpu.CompilerParams(
            dimension_semantics=("parallel",),
        ),
    )(x, codebooks)
    codes = jnp.transpose(codes_bqt, (1, 0, 2))
    commit_loss = jnp.mean(jnp.sum(loss_bq[:, 0, :], axis=0) / (_B * _T * _D))
    return q_bdt, codes, commit_loss


# speedup vs baseline: 2.3144x; 1.5086x over previous
import jax
import jax.numpy as jnp
from jax.experimental import pallas as pl
from jax.experimental.pallas import tpu as pltpu

_B, _D, _T = 16, 256, 2048
_N_Q, _BINS = 8, 1024


def _rvq_kernel(x_ref, cb_ref, q_ref, codes_ref, loss_ref):
    r = x_ref[0]
    quant = jnp.zeros_like(r)
    rn = jnp.sum(r * r, axis=0)
    tb = r.shape[1]
    losses = []
    for i in range(_N_Q):
        cb = cb_ref[i]
        cbn = jnp.sum(cb * cb, axis=1)
        s = jax.lax.dot_general(
            cb, r, (((1,), (0,)), ((), ())),
            preferred_element_type=jnp.float32)
        d = (rn[None, :] - 2.0 * s) + cbn[:, None]
        idx = jnp.argmin(d, axis=0)
        onehot = (jax.lax.broadcasted_iota(jnp.int32, (_BINS, tb), 0)
                  == idx[None, :]).astype(jnp.bfloat16)
        a = cb.astype(jnp.bfloat16)
        r1 = cb - a.astype(jnp.float32)
        bb = r1.astype(jnp.bfloat16)
        cc = (r1 - bb.astype(jnp.float32)).astype(jnp.bfloat16)
        dn = (((0,), (0,)), ((), ()))
        q = ((jax.lax.dot_general(a, onehot, dn,
                                  preferred_element_type=jnp.float32)
              + jax.lax.dot_general(bb, onehot, dn,
                                    preferred_element_type=jnp.float32))
             + jax.lax.dot_general(cc, onehot, dn,
                                   preferred_element_type=jnp.float32))
        q_st = r + (q - r)
        losses.append(jnp.sum((q - r) ** 2))
        quant = quant + q_st
        r = r - q_st
        rn = jnp.sum(r * r, axis=0)
        codes_ref[0, i, :] = idx
    q_ref[0] = quant
    loss_ref[0, 0, :] = jnp.stack(losses)


def kernel(x, codebooks):
    grid = (_B,)
    q_bdt, codes_bqt, loss_bq = pl.pallas_call(
        _rvq_kernel,
        grid=grid,
        in_specs=[
            pl.BlockSpec((1, _D, _T), lambda b: (b, 0, 0)),
            pl.BlockSpec((_N_Q, _BINS, _D), lambda b: (0, 0, 0)),
        ],
        out_specs=[
            pl.BlockSpec((1, _D, _T), lambda b: (b, 0, 0)),
            pl.BlockSpec((1, _N_Q, _T), lambda b: (b, 0, 0)),
            pl.BlockSpec((1, 1, _N_Q), lambda b: (b, 0, 0)),
        ],
        out_shape=[
            jax.ShapeDtypeStruct((_B, _D, _T), jnp.float32),
            jax.ShapeDtypeStruct((_B, _N_Q, _T), jnp.int32),
            jax.ShapeDtypeStruct((_B, 1, _N_Q), jnp.float32),
        ],
        compiler_params=pltpu.CompilerParams(
            dimension_semantics=("parallel",),
        ),
    )(x, codebooks)
    codes = jnp.transpose(codes_bqt, (1, 0, 2))
    commit_loss = jnp.mean(jnp.sum(loss_bq[:, 0, :], axis=0) / (_B * _T * _D))
    return q_bdt, codes, commit_loss
